# baseline (speedup 1.0000x reference)
import functools

import jax
import jax.numpy as jnp
from jax import lax
from jax.experimental import pallas as pl
from jax.experimental.pallas import tpu as pltpu
from jax.experimental.pallas import tpu_sc as plsc

D_MODEL = 64
SCALE = 8.0

NUM_CORES = 2
NUM_SUBCORES = 16
NUM_WORKERS = NUM_CORES * NUM_SUBCORES

ROWS_PER_GATHER = 128
GATHERS_PER_GROUP = 5
GROUP = ROWS_PER_GATHER * GATHERS_PER_GROUP
LANES = 16


@functools.partial(jax.jit, static_argnums=())
def _embed(x1d, table):
    b_total = x1d.shape[0]
    b_per_w = b_total // NUM_WORKERS
    n_groups = b_per_w // GROUP

    mesh = plsc.VectorSubcoreMesh(core_axis_name="c", subcore_axis_name="s")

    @functools.partial(
        pl.kernel,
        mesh=mesh,
        out_type=jax.ShapeDtypeStruct((b_total, D_MODEL), jnp.float32),
        compiler_params=pltpu.CompilerParams(use_tc_tiling_on_sc=False),
        scratch_types=[
            pltpu.VMEM((GROUP,), jnp.int32),
            pltpu.VMEM((GROUP, D_MODEL), jnp.float32),
            pltpu.SemaphoreType.DMA,
        ],
    )
    def body(x_hbm, table_hbm, out_hbm, idx_v, rows_v, sem):
        wid = lax.axis_index("s") * NUM_CORES + lax.axis_index("c")
        row_base = wid * b_per_w

        def group_body(g, carry):
            start = row_base + g * GROUP
            pltpu.sync_copy(x_hbm.at[pl.ds(start, GROUP)], idx_v)
            copies = [
                pltpu.async_copy(
                    table_hbm.at[
                        idx_v.at[pl.ds(j * ROWS_PER_GATHER, ROWS_PER_GATHER)]
                    ],
                    rows_v.at[pl.ds(j * ROWS_PER_GATHER, ROWS_PER_GATHER)],
                    sem,
                )
                for j in range(GATHERS_PER_GROUP)
            ]
            for cp in copies:
                cp.wait()

            def scale_row(r, c):
                for s in range(D_MODEL // LANES):
                    sl = pl.ds(s * LANES, LANES)
                    rows_v[r, sl] = rows_v[r, sl] * SCALE
                return c

            lax.fori_loop(0, GROUP, scale_row, 0, unroll=8)
            pltpu.sync_copy(rows_v, out_hbm.at[pl.ds(start, GROUP)])
            return carry

        lax.fori_loop(0, n_groups, group_body, 0)

    return body(x1d, table)


def kernel(x, table):
    batch, seq = x.shape
    b_total = batch * seq
    out = _embed(x.reshape(b_total), table)
    return out.reshape(batch, seq, D_MODEL)
